# baseline (speedup 1.0000x reference)
import functools

import jax
import jax.numpy as jnp
from jax import lax
from jax.experimental import pallas as pl
from jax.experimental.pallas import tpu as pltpu
from jax.experimental.pallas import tpu_sc as plsc

N = 50000
E = 800000
PE = 100000
D = 128
HID = 72

EPAD = 819200
EROWS = 6400
EROWS64 = 12800
PEPAD = 131072
PEROWS = 1024

HALF = 25000
ACC_ROWS = 25088
JUNK_LOCAL = 25024
QUARTER = 12500
ACC4 = 12544
JUNK_Q = 12500
CAPW = 25600
CAPC = 200
DEG_ROWS = 51200
JUNK_DEG = 50432

_SC_MESH = dict(core_axis_name="c", subcore_axis_name="s")
_SC_PARAMS = pltpu.CompilerParams(use_tc_tiling_on_sc=False)
_SC_PARAMS_NL = pltpu.CompilerParams(use_tc_tiling_on_sc=False,
                                     needs_layout_passes=False)


def _deg_body(col2, zeros1d, out, colv, ones_v, acc, sem):
    c = lax.axis_index("c")
    s = lax.axis_index("s")
    wid = s * 2 + c
    pltpu.sync_copy(zeros1d, acc.at[pl.ds(s * 3200, 3200)])
    for i in range(8):
        ones_v[pl.ds(i * 16, 16)] = jnp.ones((16,), jnp.float32)
    pltpu.sync_copy(col2.at[pl.ds(wid * 200, 200)], colv)
    plsc.subcore_barrier()

    def body(j, carry):
        pltpu.sync_copy(ones_v, acc.at[colv.at[j]], add=True)
        return carry

    lax.fori_loop(0, 200, body, 0)
    plsc.subcore_barrier()
    pltpu.sync_copy(acc.at[pl.ds(s * 3200, 3200)],
                    out.at[c, pl.ds(s * 3200, 3200)])


def _make_deg_kernel():
    return functools.partial(
        pl.kernel,
        out_type=jax.ShapeDtypeStruct((2, DEG_ROWS), jnp.float32),
        mesh=plsc.VectorSubcoreMesh(**_SC_MESH),
        compiler_params=_SC_PARAMS,
        scratch_types=[
            pltpu.VMEM((200, 128), jnp.int32),
            pltpu.VMEM((128,), jnp.float32),
            pltpu.VMEM_SHARED((DEG_ROWS,), jnp.float32),
            pltpu.SemaphoreType.DMA,
        ],
    )(_deg_body)


def _part_body(row2, col2, selr, sellc, counts, rv, cv, bufr, bufl, cntv):
    c = lax.axis_index("c")
    s = lax.axis_index("s")
    w = s * 2 + c

    for b in range(4):
        base = b * QUARTER

        def blk(o, off):
            r0 = w * 400 + o * 8
            pltpu.sync_copy(row2.at[pl.ds(r0, 8)], rv)
            pltpu.sync_copy(col2.at[pl.ds(r0, 8)], cv)
            for j in range(8):
                for i in range(4):
                    rvv = rv[j, pl.ds(i * 16, 16)]
                    lc = cv[j, pl.ds(i * 16, 16)] - base
                    m = (lc >= 0) & (lc < QUARTER)
                    mi = m.astype(jnp.int32)
                    cum = plsc.cumsum(mi)
                    pos = off + cum - mi
                    plsc.store_scatter(bufr, [pos], rvv, mask=m)
                    plsc.store_scatter(bufl, [pos], lc, mask=m)
                    off = off + jnp.max(cum)
            return off

        off = lax.fori_loop(0, 50, blk, 0)
        target = ((off + 1023) // 1024) * 1024

        def padb(k, off2):
            pos = off2 + lax.iota(jnp.int32, 16)
            plsc.store_scatter(bufr, [pos], jnp.zeros((16,), jnp.int32))
            plsc.store_scatter(bufl, [pos], jnp.full((16,), JUNK_Q, jnp.int32))
            return off2 + 16

        lax.fori_loop(0, (target - off + 15) // 16, padb, off)
        pltpu.sync_copy(bufr.at[pl.ds(0, CAPW)], selr.at[b, w])
        pltpu.sync_copy(bufl.at[pl.ds(0, CAPW)], sellc.at[b, w])
        cntv[pl.ds(0, 16)] = jnp.full((16,), target // 128, jnp.int32)
        pltpu.sync_copy(cntv, counts.at[b, w])


def _make_part_kernel():
    return functools.partial(
        pl.kernel,
        out_type=(jax.ShapeDtypeStruct((4, 32, CAPW), jnp.int32),
                  jax.ShapeDtypeStruct((4, 32, CAPW), jnp.int32),
                  jax.ShapeDtypeStruct((4, 32, 16), jnp.int32)),
        mesh=plsc.VectorSubcoreMesh(**_SC_MESH),
        compiler_params=_SC_PARAMS_NL,
        scratch_types=[
            pltpu.VMEM((8, 64), jnp.int32),
            pltpu.VMEM((8, 64), jnp.int32),
            pltpu.VMEM((CAPW + 16,), jnp.int32),
            pltpu.VMEM((CAPW + 16,), jnp.int32),
            pltpu.VMEM((16,), jnp.int32),
        ],
    )(_part_body)


def _spmm_body(t_hbm, row2, col2, zeros2d, outp,
               row_v, col_v, rows_v, acc, sem):
    c = lax.axis_index("c")
    s = lax.axis_index("s")
    base = c * HALF
    pltpu.sync_copy(zeros2d, acc.at[pl.ds(s * 1568, 1568)])
    plsc.subcore_barrier()

    def chunk_body(o, carry):
        r0 = s * 400 + o * 16
        pltpu.sync_copy(row2.at[pl.ds(r0, 16)], row_v)
        pltpu.sync_copy(col2.at[pl.ds(r0, 16)], col_v)

        def body(j, carry2):
            for i in range(8):
                cv = col_v[j, pl.ds(i * 16, 16)]
                rv = row_v[j, pl.ds(i * 16, 16)]
                lc = cv - base
                ok = (lc >= 0) & (lc < HALF)
                col_v[j, pl.ds(i * 16, 16)] = jnp.where(ok, lc, JUNK_LOCAL)
                row_v[j, pl.ds(i * 16, 16)] = jnp.where(ok, rv, 0)
            pltpu.async_copy(t_hbm.at[row_v.at[j]], rows_v, sem).wait()
            pltpu.sync_copy(rows_v, acc.at[col_v.at[j]], add=True)
            return carry2

        lax.fori_loop(0, 16, body, 0)
        return carry

    lax.fori_loop(0, 25, chunk_body, 0)
    plsc.subcore_barrier()
    pltpu.sync_copy(acc.at[pl.ds(s * 1568, 1568)],
                    outp.at[c, pl.ds(s * 1568, 1568)])


def _make_spmm_kernel():
    return functools.partial(
        pl.kernel,
        out_type=jax.ShapeDtypeStruct((2, ACC_ROWS, HID), jnp.float32),
        mesh=plsc.VectorSubcoreMesh(**_SC_MESH),
        compiler_params=_SC_PARAMS,
        scratch_types=[
            pltpu.VMEM((16, 128), jnp.int32),
            pltpu.VMEM((16, 128), jnp.int32),
            pltpu.VMEM((128, HID), jnp.float32),
            pltpu.VMEM_SHARED((ACC_ROWS, HID), jnp.float32),
            pltpu.SemaphoreType.DMA,
        ],
    )(_spmm_body)


def _pedge_body(u_hbm, v_hbm, src2, dst2, outU, outV, si, di, ub, vb,
                semu, semv):
    c = lax.axis_index("c")
    s = lax.axis_index("s")
    wid = s * 2 + c

    @pl.when(wid < 16)
    def _():
        pltpu.sync_copy(src2.at[pl.ds(wid * 64, 64)], si)
        pltpu.sync_copy(dst2.at[pl.ds(wid * 64, 64)], di)

        def body(j, carry):
            pltpu.async_copy(u_hbm.at[si.at[j]], ub, semu).wait()
            pltpu.sync_copy(ub, outU.at[pl.ds(wid * 8192 + j * 128, 128)])
            pltpu.async_copy(v_hbm.at[di.at[j]], vb, semv).wait()
            pltpu.sync_copy(vb, outV.at[pl.ds(wid * 8192 + j * 128, 128)])
            return carry

        lax.fori_loop(0, 64, body, 0)


def _make_pedge_kernel():
    return functools.partial(
        pl.kernel,
        out_type=(jax.ShapeDtypeStruct((PEPAD, HID), jnp.float32),
                  jax.ShapeDtypeStruct((PEPAD, HID), jnp.float32)),
        mesh=plsc.VectorSubcoreMesh(**_SC_MESH),
        compiler_params=_SC_PARAMS,
        scratch_types=[
            pltpu.VMEM((64, 128), jnp.int32),
            pltpu.VMEM((64, 128), jnp.int32),
            pltpu.VMEM((128, HID), jnp.float32),
            pltpu.VMEM((128, HID), jnp.float32),
            pltpu.SemaphoreType.DMA,
            pltpu.SemaphoreType.DMA,
        ],
    )(_pedge_body)


def _enc_kernel(x_ref, emb_ref, wn_ref, bn_ref, we_ref, be_ref, h_ref, e_ref):
    h_ref[...] = jnp.maximum(x_ref[...] @ wn_ref[...] + bn_ref[...], 0.0)
    e_ref[...] = jnp.maximum(emb_ref[...] @ we_ref[...] + be_ref[...], 0.0)


def _dinv_kernel(p_ref, o_ref):
    dsum = p_ref[0] + p_ref[1]
    o_ref[...] = jnp.where(dsum > 0.0,
                           lax.rsqrt(jnp.maximum(dsum, 1e-12)), 0.0)


def _pre_kernel(h_ref, dinv_ref, wi_ref, wr_ref, t_ref, r_ref):
    hs = h_ref[...] * dinv_ref[...]
    t_ref[...] = hs @ wi_ref[...]
    r_ref[...] = h_ref[...] @ wr_ref[...]


def _post_kernel(agg_ref, dinv_ref, r_ref, bias_ref, out_ref, sums_ref):
    b = pl.program_id(0)
    o = jnp.maximum(agg_ref[0] * dinv_ref[...] + r_ref[...] + bias_ref[...],
                    0.0)
    out_ref[...] = o
    part = jnp.stack([jnp.sum(o, axis=0), jnp.sum(o * o, axis=0)])

    @pl.when(b == 0)
    def _():
        sums_ref[...] = part

    @pl.when(b > 0)
    def _():
        sums_ref[...] += part


def _bn_kernel(out_ref, sums_ref, gamma_ref, beta_ref, h_ref):
    inv_n = 1.0 / N
    mean = sums_ref[0, :] * inv_n
    var = sums_ref[1, :] * inv_n - mean * mean
    scale = lax.rsqrt(var + 1e-5) * gamma_ref[0]
    h_ref[...] = (out_ref[...] - mean) * scale + beta_ref[0]


def _uv_kernel(h_ref, e_ref, wa_ref, wb_ref, wc_ref, wd_ref, b1_ref,
               u_ref, v_ref):
    u_ref[...] = (h_ref[...] @ wa_ref[...] + e_ref[...] @ wb_ref[...]
                  + b1_ref[...])
    v_ref[...] = h_ref[...] @ wc_ref[...] + e_ref[...] @ wd_ref[...]


def _fin_kernel(u_ref, v_ref, w2_ref, b2_ref, y_ref):
    y_ref[...] = jnp.tanh(u_ref[...] + v_ref[...]) @ w2_ref[...] + b2_ref[0]


def _full(shape):
    nd = len(shape)
    return pl.BlockSpec(shape, lambda b: (0,) * nd)


def kernel(x, emb, edge_index, pedge_index, W_node, b_node, W_emb, b_emb,
           conv_init_w, conv_root_w, conv_bias, bn_gamma, bn_beta,
           W1, b1, W2, b2):
    f32 = jnp.float32
    row = edge_index[0]
    col = edge_index[1]
    row2 = jnp.pad(row, (0, EPAD - E)).reshape(EROWS, 128)
    col2 = jnp.pad(col, (0, EPAD - E),
                   constant_values=JUNK_DEG).reshape(EROWS, 128)
    src2 = jnp.pad(pedge_index[0], (0, PEPAD - PE)).reshape(PEROWS, 128)
    dst2 = jnp.pad(pedge_index[1], (0, PEPAD - PE)).reshape(PEROWS, 128)

    zeros1d = jnp.zeros((3200,), f32)
    zeros2d = jnp.zeros((1568, HID), f32)

    Wa = W1[0:72] + W1[144:216] + W1[288:360]
    Wb = W1[72:144] + W1[216:288] + W1[360:432]
    Wc = W1[432:504] + W1[576:648] + W1[720:792]
    Wd = W1[504:576] + W1[648:720] + W1[792:864]

    bn1 = b_node.reshape(1, HID)
    be1 = b_emb.reshape(1, HID)
    b1r = b1.reshape(1, HID)
    b2r = b2.reshape(1, 1)

    deg_parts = _make_deg_kernel()(col2, zeros1d)

    grid25 = 25
    BLK = 2000
    h, e = pl.pallas_call(
        _enc_kernel,
        grid=(grid25,),
        in_specs=[
            pl.BlockSpec((BLK, D), lambda b: (b, 0)),
            pl.BlockSpec((BLK, D), lambda b: (b, 0)),
            _full((D, HID)), _full((1, HID)),
            _full((D, HID)), _full((1, HID)),
        ],
        out_specs=[
            pl.BlockSpec((BLK, HID), lambda b: (b, 0)),
            pl.BlockSpec((BLK, HID), lambda b: (b, 0)),
        ],
        out_shape=[
            jax.ShapeDtypeStruct((N, HID), f32),
            jax.ShapeDtypeStruct((N, HID), f32),
        ],
    )(x, emb, W_node, bn1, W_emb, be1)

    dinv2d = pl.pallas_call(
        _dinv_kernel,
        out_shape=jax.ShapeDtypeStruct((400, 128), f32),
    )(deg_parts.reshape(2, 400, 128))
    dinv = dinv2d.reshape(DEG_ROWS, 1)[:N]

    spmm = _make_spmm_kernel()

    for l in range(3):
        wi = conv_init_w[l]
        wr = conv_root_w[l]
        bias = conv_bias[l].reshape(1, HID)
        t, r = pl.pallas_call(
            _pre_kernel,
            grid=(grid25,),
            in_specs=[
                pl.BlockSpec((BLK, HID), lambda b: (b, 0)),
                pl.BlockSpec((BLK, 1), lambda b: (b, 0)),
                _full((HID, HID)), _full((HID, HID)),
            ],
            out_specs=[
                pl.BlockSpec((BLK, HID), lambda b: (b, 0)),
                pl.BlockSpec((BLK, HID), lambda b: (b, 0)),
            ],
            out_shape=[
                jax.ShapeDtypeStruct((N, HID), f32),
                jax.ShapeDtypeStruct((N, HID), f32),
            ],
        )(h, dinv, wi, wr)

        agg_parts = spmm(t, row2, col2, zeros2d)

        out, sums = pl.pallas_call(
            _post_kernel,
            grid=(50,),
            in_specs=[
                pl.BlockSpec((1, 1000, HID), lambda b: (b // 25, b % 25, 0)),
                pl.BlockSpec((1000, 1), lambda b: (b, 0)),
                pl.BlockSpec((1000, HID), lambda b: (b, 0)),
                _full((1, HID)),
            ],
            out_specs=[
                pl.BlockSpec((1000, HID), lambda b: (b, 0)),
                pl.BlockSpec((2, HID), lambda b: (0, 0)),
            ],
            out_shape=[
                jax.ShapeDtypeStruct((N, HID), f32),
                jax.ShapeDtypeStruct((2, HID), f32),
            ],
        )(agg_parts, dinv, r, bias)

        h = pl.pallas_call(
            _bn_kernel,
            grid=(grid25,),
            in_specs=[
                pl.BlockSpec((BLK, HID), lambda b: (b, 0)),
                _full((2, HID)), _full((1, HID)), _full((1, HID)),
            ],
            out_specs=pl.BlockSpec((BLK, HID), lambda b: (b, 0)),
            out_shape=jax.ShapeDtypeStruct((N, HID), f32),
        )(out, sums, bn_gamma[l].reshape(1, HID), bn_beta[l].reshape(1, HID))

    u, v = pl.pallas_call(
        _uv_kernel,
        grid=(grid25,),
        in_specs=[
            pl.BlockSpec((BLK, HID), lambda b: (b, 0)),
            pl.BlockSpec((BLK, HID), lambda b: (b, 0)),
            _full((HID, HID)), _full((HID, HID)),
            _full((HID, HID)), _full((HID, HID)),
            _full((1, HID)),
        ],
        out_specs=[
            pl.BlockSpec((BLK, HID), lambda b: (b, 0)),
            pl.BlockSpec((BLK, HID), lambda b: (b, 0)),
        ],
        out_shape=[
            jax.ShapeDtypeStruct((N, HID), f32),
            jax.ShapeDtypeStruct((N, HID), f32),
        ],
    )(h, e, Wa, Wb, Wc, Wd, b1r)

    U, V = _make_pedge_kernel()(u, v, src2, dst2)

    y = pl.pallas_call(
        _fin_kernel,
        grid=(32,),
        in_specs=[
            pl.BlockSpec((4096, HID), lambda b: (b, 0)),
            pl.BlockSpec((4096, HID), lambda b: (b, 0)),
            _full((HID, 1)), _full((1, 1)),
        ],
        out_specs=pl.BlockSpec((4096, 1), lambda b: (b, 0)),
        out_shape=jax.ShapeDtypeStruct((PEPAD, 1), f32),
    )(U, V, W2, b2r)

    return y[:PE]


# speedup vs baseline: 20.6440x; 20.6440x over previous
import functools

import jax
import jax.numpy as jnp
from jax import lax
from jax.experimental import pallas as pl
from jax.experimental.pallas import tpu as pltpu
from jax.experimental.pallas import tpu_sc as plsc

N = 50000
E = 800000
PE = 100000
D = 128
HID = 72

EPAD = 819200
EROWS = 6400
EROWS64 = 12800
PEPAD = 102400
PEROWS = 800

HALF = 25000
ACC_ROWS = 25088
JUNK_LOCAL = 25024
QUARTER = 12500
ACC4 = 12544
JUNK_Q = 12500
CAPW = 25600
CAPC = 200
DEG_ROWS = 51200
JUNK_DEG = 50432

_SC_MESH = dict(core_axis_name="c", subcore_axis_name="s")
_SC_PARAMS = pltpu.CompilerParams(use_tc_tiling_on_sc=False)
_SC_PARAMS_NL = pltpu.CompilerParams(use_tc_tiling_on_sc=False,
                                     needs_layout_passes=False)


def _deg_body(col2, zeros1d, out, colv, ones_v, acc, sem):
    c = lax.axis_index("c")
    s = lax.axis_index("s")
    wid = s * 2 + c
    pltpu.sync_copy(zeros1d, acc.at[pl.ds(s * 3200, 3200)])
    for i in range(8):
        ones_v[pl.ds(i * 16, 16)] = jnp.ones((16,), jnp.float32)
    pltpu.sync_copy(col2.at[pl.ds(wid * 200, 200)], colv)
    plsc.subcore_barrier()

    def body(j, carry):
        pltpu.sync_copy(ones_v, acc.at[colv.at[j]], add=True)
        return carry

    lax.fori_loop(0, 200, body, 0)
    plsc.subcore_barrier()
    pltpu.sync_copy(acc.at[pl.ds(s * 3200, 3200)],
                    out.at[c, pl.ds(s * 3200, 3200)])


def _make_deg_kernel():
    return functools.partial(
        pl.kernel,
        out_type=jax.ShapeDtypeStruct((2, DEG_ROWS), jnp.float32),
        mesh=plsc.VectorSubcoreMesh(**_SC_MESH),
        compiler_params=_SC_PARAMS,
        scratch_types=[
            pltpu.VMEM((200, 128), jnp.int32),
            pltpu.VMEM((128,), jnp.float32),
            pltpu.VMEM_SHARED((DEG_ROWS,), jnp.float32),
            pltpu.SemaphoreType.DMA,
        ],
    )(_deg_body)


def _part_body(row2, col2, selr, sellc, counts, rv, cv, bufr, bufl, cntv):
    c = lax.axis_index("c")
    s = lax.axis_index("s")
    w = s * 2 + c

    for b in range(4):
        base = b * QUARTER

        def blk(o, off):
            r0 = w * 400 + o * 8
            pltpu.sync_copy(row2.at[pl.ds(r0, 8)], rv)
            pltpu.sync_copy(col2.at[pl.ds(r0, 8)], cv)
            for j in range(8):
                for i in range(4):
                    rvv = rv[j, pl.ds(i * 16, 16)]
                    lc = cv[j, pl.ds(i * 16, 16)] - base
                    m = (lc >= 0) & (lc < QUARTER)
                    mi = m.astype(jnp.int32)
                    cum = plsc.cumsum(mi)
                    pos = off + cum - mi
                    plsc.store_scatter(bufr, [pos], rvv, mask=m)
                    plsc.store_scatter(bufl, [pos], lc, mask=m)
                    off = off + jnp.max(cum)
            return off

        off = lax.fori_loop(0, 50, blk, 0)
        target = ((off + 1023) // 1024) * 1024

        def padb(k, off2):
            pos = off2 + lax.iota(jnp.int32, 16)
            plsc.store_scatter(bufr, [pos], jnp.zeros((16,), jnp.int32))
            plsc.store_scatter(bufl, [pos], jnp.full((16,), JUNK_Q, jnp.int32))
            return off2 + 16

        lax.fori_loop(0, (target - off + 15) // 16, padb, off)
        pltpu.sync_copy(bufr.at[pl.ds(0, CAPW)], selr.at[b, w])
        pltpu.sync_copy(bufl.at[pl.ds(0, CAPW)], sellc.at[b, w])
        cntv[pl.ds(0, 16)] = jnp.full((16,), target // 128, jnp.int32)
        pltpu.sync_copy(cntv, counts.at[b, w])


def _make_part_kernel():
    return functools.partial(
        pl.kernel,
        out_type=(jax.ShapeDtypeStruct((4, 32, CAPW), jnp.int32),
                  jax.ShapeDtypeStruct((4, 32, CAPW), jnp.int32),
                  jax.ShapeDtypeStruct((4, 32, 16), jnp.int32)),
        mesh=plsc.VectorSubcoreMesh(**_SC_MESH),
        compiler_params=_SC_PARAMS_NL,
        scratch_types=[
            pltpu.VMEM((8, 64), jnp.int32),
            pltpu.VMEM((8, 64), jnp.int32),
            pltpu.VMEM((CAPW + 16,), jnp.int32),
            pltpu.VMEM((CAPW + 16,), jnp.int32),
            pltpu.VMEM((16,), jnp.int32),
        ],
    )(_part_body)


def _spmm_body(t_hbm, row2, col2, zeros2d, outp,
               row_v, col_v, rows_v, acc, sem):
    c = lax.axis_index("c")
    s = lax.axis_index("s")
    base = c * HALF
    pltpu.sync_copy(zeros2d, acc.at[pl.ds(s * 1568, 1568)])
    plsc.subcore_barrier()

    def chunk_body(o, carry):
        r0 = s * 400 + o * 16
        pltpu.sync_copy(row2.at[pl.ds(r0, 16)], row_v)
        pltpu.sync_copy(col2.at[pl.ds(r0, 16)], col_v)

        def body(j, carry2):
            for i in range(8):
                cv = col_v[j, pl.ds(i * 16, 16)]
                lc = cv - base
                ok = (lc >= 0) & (lc < HALF)
                junk = JUNK_LOCAL + (cv & 63)
                col_v[j, pl.ds(i * 16, 16)] = jnp.where(ok, lc, junk)
            pltpu.async_copy(t_hbm.at[row_v.at[j]], rows_v, sem).wait()
            pltpu.sync_copy(rows_v, acc.at[col_v.at[j]], add=True)
            return carry2

        lax.fori_loop(0, 16, body, 0)
        return carry

    lax.fori_loop(0, 25, chunk_body, 0)
    plsc.subcore_barrier()
    pltpu.sync_copy(acc.at[pl.ds(s * 1568, 1568)],
                    outp.at[c, pl.ds(s * 1568, 1568)])


def _make_spmm_kernel():
    return functools.partial(
        pl.kernel,
        out_type=jax.ShapeDtypeStruct((2, ACC_ROWS, HID), jnp.float32),
        mesh=plsc.VectorSubcoreMesh(**_SC_MESH),
        compiler_params=_SC_PARAMS,
        scratch_types=[
            pltpu.VMEM((16, 128), jnp.int32),
            pltpu.VMEM((16, 128), jnp.int32),
            pltpu.VMEM((128, HID), jnp.float32),
            pltpu.VMEM_SHARED((ACC_ROWS, HID), jnp.float32),
            pltpu.SemaphoreType.DMA,
        ],
    )(_spmm_body)


def _pedge_body(u_hbm, v_hbm, src2, dst2, outU, outV, si, di, ub, vb,
                semu, semv):
    c = lax.axis_index("c")
    s = lax.axis_index("s")
    wid = s * 2 + c

    @pl.when(wid < 25)
    def _():
        pltpu.sync_copy(src2.at[pl.ds(wid * 32, 32)], si)
        pltpu.sync_copy(dst2.at[pl.ds(wid * 32, 32)], di)

        def body(j, carry):
            pltpu.async_copy(u_hbm.at[si.at[j]], ub, semu).wait()
            pltpu.sync_copy(ub, outU.at[pl.ds(wid * 4096 + j * 128, 128)])
            pltpu.async_copy(v_hbm.at[di.at[j]], vb, semv).wait()
            pltpu.sync_copy(vb, outV.at[pl.ds(wid * 4096 + j * 128, 128)])
            return carry

        lax.fori_loop(0, 32, body, 0)


def _make_pedge_kernel():
    return functools.partial(
        pl.kernel,
        out_type=(jax.ShapeDtypeStruct((PEPAD, HID), jnp.float32),
                  jax.ShapeDtypeStruct((PEPAD, HID), jnp.float32)),
        mesh=plsc.VectorSubcoreMesh(**_SC_MESH),
        compiler_params=_SC_PARAMS,
        scratch_types=[
            pltpu.VMEM((32, 128), jnp.int32),
            pltpu.VMEM((32, 128), jnp.int32),
            pltpu.VMEM((128, HID), jnp.float32),
            pltpu.VMEM((128, HID), jnp.float32),
            pltpu.SemaphoreType.DMA,
            pltpu.SemaphoreType.DMA,
        ],
    )(_pedge_body)


def _enc_kernel(x_ref, emb_ref, wn_ref, bn_ref, we_ref, be_ref, h_ref, e_ref):
    h_ref[...] = jnp.maximum(x_ref[...] @ wn_ref[...] + bn_ref[...], 0.0)
    e_ref[...] = jnp.maximum(emb_ref[...] @ we_ref[...] + be_ref[...], 0.0)


def _dinv_kernel(p_ref, o_ref):
    dsum = p_ref[0] + p_ref[1]
    o_ref[...] = jnp.where(dsum > 0.0,
                           lax.rsqrt(jnp.maximum(dsum, 1e-12)), 0.0)


def _pre_kernel(h_ref, dinv_ref, wi_ref, wr_ref, t_ref, r_ref):
    hs = h_ref[...] * dinv_ref[...]
    t_ref[...] = hs @ wi_ref[...]
    r_ref[...] = h_ref[...] @ wr_ref[...]


def _post_kernel(agg_ref, dinv_ref, r_ref, bias_ref, out_ref, sums_ref):
    b = pl.program_id(0)
    o = jnp.maximum(agg_ref[0] * dinv_ref[...] + r_ref[...] + bias_ref[...],
                    0.0)
    out_ref[...] = o
    part = jnp.stack([jnp.sum(o, axis=0), jnp.sum(o * o, axis=0)])

    @pl.when(b == 0)
    def _():
        sums_ref[...] = part

    @pl.when(b > 0)
    def _():
        sums_ref[...] += part


def _bn_kernel(out_ref, sums_ref, gamma_ref, beta_ref, h_ref):
    inv_n = 1.0 / N
    mean = sums_ref[0, :] * inv_n
    var = sums_ref[1, :] * inv_n - mean * mean
    scale = lax.rsqrt(var + 1e-5) * gamma_ref[0]
    h_ref[...] = (out_ref[...] - mean) * scale + beta_ref[0]


def _uv_kernel(h_ref, e_ref, wa_ref, wb_ref, wc_ref, wd_ref, b1_ref,
               u_ref, v_ref):
    u_ref[...] = (h_ref[...] @ wa_ref[...] + e_ref[...] @ wb_ref[...]
                  + b1_ref[...])
    v_ref[...] = h_ref[...] @ wc_ref[...] + e_ref[...] @ wd_ref[...]


def _fin_kernel(u_ref, v_ref, w2_ref, b2_ref, y_ref):
    y_ref[...] = jnp.tanh(u_ref[...] + v_ref[...]) @ w2_ref[...] + b2_ref[0]


def _full(shape):
    nd = len(shape)
    return pl.BlockSpec(shape, lambda b: (0,) * nd)


def kernel(x, emb, edge_index, pedge_index, W_node, b_node, W_emb, b_emb,
           conv_init_w, conv_root_w, conv_bias, bn_gamma, bn_beta,
           W1, b1, W2, b2):
    f32 = jnp.float32
    row = edge_index[0]
    col = edge_index[1]
    i32 = jnp.int32
    epad_fill_row = (jnp.arange(EPAD - E, dtype=i32) * 911) % N
    epad_fill_col = JUNK_DEG + (jnp.arange(EPAD - E, dtype=i32) % 768)
    row2 = jnp.concatenate([row, epad_fill_row]).reshape(EROWS, 128)
    col2 = jnp.concatenate([col, epad_fill_col]).reshape(EROWS, 128)
    pepad_fill = (jnp.arange(PEPAD - PE, dtype=i32) * 877) % N
    src2 = jnp.concatenate([pedge_index[0], pepad_fill]).reshape(PEROWS, 128)
    dst2 = jnp.concatenate([pedge_index[1], pepad_fill]).reshape(PEROWS, 128)

    zeros1d = jnp.zeros((3200,), f32)
    zeros2d = jnp.zeros((1568, HID), f32)

    Wa = W1[0:72] + W1[144:216] + W1[288:360]
    Wb = W1[72:144] + W1[216:288] + W1[360:432]
    Wc = W1[432:504] + W1[576:648] + W1[720:792]
    Wd = W1[504:576] + W1[648:720] + W1[792:864]

    bn1 = b_node.reshape(1, HID)
    be1 = b_emb.reshape(1, HID)
    b1r = b1.reshape(1, HID)
    b2r = b2.reshape(1, 1)

    deg_parts = _make_deg_kernel()(col2, zeros1d)

    grid25 = 25
    BLK = 2000
    h, e = pl.pallas_call(
        _enc_kernel,
        grid=(grid25,),
        in_specs=[
            pl.BlockSpec((BLK, D), lambda b: (b, 0)),
            pl.BlockSpec((BLK, D), lambda b: (b, 0)),
            _full((D, HID)), _full((1, HID)),
            _full((D, HID)), _full((1, HID)),
        ],
        out_specs=[
            pl.BlockSpec((BLK, HID), lambda b: (b, 0)),
            pl.BlockSpec((BLK, HID), lambda b: (b, 0)),
        ],
        out_shape=[
            jax.ShapeDtypeStruct((N, HID), f32),
            jax.ShapeDtypeStruct((N, HID), f32),
        ],
    )(x, emb, W_node, bn1, W_emb, be1)

    dinv2d = pl.pallas_call(
        _dinv_kernel,
        out_shape=jax.ShapeDtypeStruct((400, 128), f32),
    )(deg_parts.reshape(2, 400, 128))
    dinv = dinv2d.reshape(DEG_ROWS, 1)[:N]

    spmm = _make_spmm_kernel()

    for l in range(3):
        wi = conv_init_w[l]
        wr = conv_root_w[l]
        bias = conv_bias[l].reshape(1, HID)
        t, r = pl.pallas_call(
            _pre_kernel,
            grid=(grid25,),
            in_specs=[
                pl.BlockSpec((BLK, HID), lambda b: (b, 0)),
                pl.BlockSpec((BLK, 1), lambda b: (b, 0)),
                _full((HID, HID)), _full((HID, HID)),
            ],
            out_specs=[
                pl.BlockSpec((BLK, HID), lambda b: (b, 0)),
                pl.BlockSpec((BLK, HID), lambda b: (b, 0)),
            ],
            out_shape=[
                jax.ShapeDtypeStruct((N, HID), f32),
                jax.ShapeDtypeStruct((N, HID), f32),
            ],
        )(h, dinv, wi, wr)

        agg_parts = spmm(t, row2, col2, zeros2d)

        out, sums = pl.pallas_call(
            _post_kernel,
            grid=(50,),
            in_specs=[
                pl.BlockSpec((1, 1000, HID), lambda b: (b // 25, b % 25, 0)),
                pl.BlockSpec((1000, 1), lambda b: (b, 0)),
                pl.BlockSpec((1000, HID), lambda b: (b, 0)),
                _full((1, HID)),
            ],
            out_specs=[
                pl.BlockSpec((1000, HID), lambda b: (b, 0)),
                pl.BlockSpec((2, HID), lambda b: (0, 0)),
            ],
            out_shape=[
                jax.ShapeDtypeStruct((N, HID), f32),
                jax.ShapeDtypeStruct((2, HID), f32),
            ],
        )(agg_parts, dinv, r, bias)

        h = pl.pallas_call(
            _bn_kernel,
            grid=(grid25,),
            in_specs=[
                pl.BlockSpec((BLK, HID), lambda b: (b, 0)),
                _full((2, HID)), _full((1, HID)), _full((1, HID)),
            ],
            out_specs=pl.BlockSpec((BLK, HID), lambda b: (b, 0)),
            out_shape=jax.ShapeDtypeStruct((N, HID), f32),
        )(out, sums, bn_gamma[l].reshape(1, HID), bn_beta[l].reshape(1, HID))

    u, v = pl.pallas_call(
        _uv_kernel,
        grid=(grid25,),
        in_specs=[
            pl.BlockSpec((BLK, HID), lambda b: (b, 0)),
            pl.BlockSpec((BLK, HID), lambda b: (b, 0)),
            _full((HID, HID)), _full((HID, HID)),
            _full((HID, HID)), _full((HID, HID)),
            _full((1, HID)),
        ],
        out_specs=[
            pl.BlockSpec((BLK, HID), lambda b: (b, 0)),
            pl.BlockSpec((BLK, HID), lambda b: (b, 0)),
        ],
        out_shape=[
            jax.ShapeDtypeStruct((N, HID), f32),
            jax.ShapeDtypeStruct((N, HID), f32),
        ],
    )(h, e, Wa, Wb, Wc, Wd, b1r)

    U, V = _make_pedge_kernel()(u, v, src2, dst2)

    y = pl.pallas_call(
        _fin_kernel,
        grid=(32,),
        in_specs=[
            pl.BlockSpec((3200, HID), lambda b: (b, 0)),
            pl.BlockSpec((3200, HID), lambda b: (b, 0)),
            _full((HID, 1)), _full((1, 1)),
        ],
        out_specs=pl.BlockSpec((3200, 1), lambda b: (b, 0)),
        out_shape=jax.ShapeDtypeStruct((PEPAD, 1), f32),
    )(U, V, W2, b2r)

    return y[:PE]
